# baseline (speedup 1.0000x reference)
import functools

import jax
import jax.numpy as jnp
from jax import lax
from jax.experimental import pallas as pl
from jax.experimental.pallas import tpu as pltpu
from jax.experimental.pallas import tpu_sc as plsc

TOPK = 32
LANES = 16
NUM_CORES = 2
NUM_SUBCORES = 16
NUM_WORKERS = NUM_CORES * NUM_SUBCORES
HIST_BUCKETS = 256


def _to_ord(xv):
    b = lax.bitcast_convert_type(xv, jnp.int32)
    flip = (b >> 31) | jnp.int32(-2147483648)
    return lax.bitcast_convert_type(b ^ flip, jnp.uint32)


def _make_topk_kernel(nrows, ncols):
    rows_per_worker = nrows // NUM_WORKERS
    nchunks = ncols // LANES
    mesh = plsc.VectorSubcoreMesh(core_axis_name="c", subcore_axis_name="s")

    @functools.partial(
        pl.kernel,
        mesh=mesh,
        compiler_params=pltpu.CompilerParams(needs_layout_passes=False),
        out_type=jax.ShapeDtypeStruct((nrows, ncols), jnp.float32),
        scratch_types=[
            pltpu.VMEM((ncols,), jnp.float32),
            pltpu.VMEM((LANES * HIST_BUCKETS,), jnp.int32),
        ],
    )
    def topk_kernel(x_hbm, out_hbm, row_v, hist_v):
        wid = lax.axis_index("s") * NUM_CORES + lax.axis_index("c")
        lane_iota = lax.iota(jnp.int32, LANES)
        lane_base = lane_iota * HIST_BUCKETS
        ones_i = jnp.ones((LANES,), jnp.int32)
        zeros_f = jnp.zeros((LANES,), jnp.float32)

        def process_row(r):
            pltpu.sync_copy(x_hbm.at[r], row_v)

            def round_body(rnd, carry):
                pm, pb, krem = carry
                shift = (jnp.uint32(24) - jnp.uint32(8) * rnd.astype(jnp.uint32))

                def zero_body(i, _):
                    hist_v[pl.ds(i * LANES, LANES)] = jnp.zeros(
                        (LANES,), jnp.int32)
                    return 0
                lax.fori_loop(0, (LANES * HIST_BUCKETS) // LANES, zero_body, 0)

                def hist_body(i, _):
                    xv = row_v[pl.ds(i * LANES, LANES)]
                    u = _to_ord(xv)
                    matc = (u & pm) == pb
                    digit = ((u >> shift) & jnp.uint32(0xFF)).astype(jnp.int32)
                    plsc.addupdate_scatter(
                        hist_v, [lane_base + digit], ones_i, mask=matc)
                    return 0
                lax.fori_loop(0, nchunks, hist_body, 0)

                def find_body(j, fc):
                    acc, d_found, ca = fc
                    c = 15 - j

                    def lsum(l, tot):
                        return tot + hist_v[
                            pl.ds(l * HIST_BUCKETS + c * LANES, LANES)]
                    totals = lax.fori_loop(
                        0, LANES, lsum, jnp.zeros((LANES,), jnp.int32))
                    rev = jnp.flip(totals)
                    cs = jnp.cumsum(rev)
                    cum = cs + acc
                    cond = cum >= krem
                    j0 = jnp.min(jnp.where(cond, lane_iota, jnp.int32(16)))
                    excl = acc + jnp.sum(
                        jnp.where(lane_iota < j0, rev, jnp.int32(0)))
                    d_c = c * 16 + 15 - j0
                    newly = jnp.logical_and(d_found < 0, j0 < 16)
                    d_found = jnp.where(newly, d_c, d_found)
                    ca = jnp.where(newly, excl, ca)
                    acc = acc + jnp.sum(totals)
                    return acc, d_found, ca

                _, d_found, ca = lax.fori_loop(
                    0, 16, find_body,
                    (jnp.int32(0), jnp.int32(-1), jnp.int32(0)))

                d_u = d_found.astype(jnp.uint32)
                pb = pb | (d_u << shift)
                pm = pm | (jnp.uint32(0xFF) << shift)
                krem = krem - ca
                return pm, pb, krem

            _, t, m = lax.fori_loop(
                0, 4, round_body,
                (jnp.uint32(0), jnp.uint32(0), jnp.int32(TOPK)))

            def out_body(i, eqrun):
                xv = row_v[pl.ds(i * LANES, LANES)]
                u = _to_ord(xv)
                gt = u > t
                eq = u == t
                eqi = eq.astype(jnp.int32)
                cs = jnp.cumsum(eqi)
                keep = gt | jnp.logical_and(eq, (cs + eqrun) <= m)
                row_v[pl.ds(i * LANES, LANES)] = jnp.where(keep, xv, zeros_f)
                return eqrun + jnp.max(cs)
            lax.fori_loop(0, nchunks, out_body, jnp.int32(0))

            pltpu.sync_copy(row_v, out_hbm.at[r])

        for rr in range(rows_per_worker):
            process_row(wid * rows_per_worker + rr)

    return topk_kernel


@jax.jit
def kernel(x):
    nrows, ncols = x.shape
    return _make_topk_kernel(nrows, ncols)(x)


# speedup vs baseline: 1.7712x; 1.7712x over previous
import functools

import jax
import jax.numpy as jnp
from jax import lax
from jax.experimental import pallas as pl
from jax.experimental.pallas import tpu as pltpu
from jax.experimental.pallas import tpu_sc as plsc

TOPK = 32
LANES = 16
NUM_CORES = 2
NUM_SUBCORES = 16
NUM_WORKERS = NUM_CORES * NUM_SUBCORES
HIST8 = 256
HIST4 = 16
CAP = 4096

def _to_ord(xv):
    b = lax.bitcast_convert_type(xv, jnp.int32)
    flip = (b >> 31) | jnp.int32(-2147483648)
    return lax.bitcast_convert_type(b ^ flip, jnp.uint32)


def _from_ord(u):
    ui = lax.bitcast_convert_type(u, jnp.int32)
    flip = ((~ui) >> 31) | jnp.int32(-2147483648)
    return lax.bitcast_convert_type(ui ^ flip, jnp.float32)


def _make_topk_kernel(nrows, ncols):
    rows_per_worker = nrows // NUM_WORKERS
    nchunks = ncols // LANES
    mesh = plsc.VectorSubcoreMesh(core_axis_name="c", subcore_axis_name="s")

    @functools.partial(
        pl.kernel,
        mesh=mesh,
        compiler_params=pltpu.CompilerParams(needs_layout_passes=False),
        out_type=jax.ShapeDtypeStruct((nrows, ncols), jnp.float32),
        scratch_types=[
            pltpu.VMEM((ncols,), jnp.float32),
            pltpu.VMEM((ncols,), jnp.float32),
            pltpu.VMEM((LANES * HIST8,), jnp.int32),
            pltpu.VMEM((CAP + LANES,), jnp.uint32),
            pltpu.VMEM((CAP + LANES,), jnp.int32),
        ],
    )
    def topk_kernel(x_hbm, out_hbm, row_v, out_v, hist_v, cand_u, cand_i):
        wid = lax.axis_index("s") * NUM_CORES + lax.axis_index("c")
        lane_iota = lax.iota(jnp.int32, LANES)
        lane_base8 = lane_iota * HIST8
        lane_base4 = lane_iota * HIST4
        ones_i = jnp.ones((LANES,), jnp.int32)
        zeros_i = jnp.zeros((LANES,), jnp.int32)
        zeros_f = jnp.zeros((LANES,), jnp.float32)

        def zero_hist(nwords):
            def zb(i, _):
                hist_v[pl.ds(i * LANES, LANES)] = zeros_i
                return 0
            lax.fori_loop(0, nwords // LANES, zb, 0)

        def find8(krem):
            def fb(j, fc):
                acc, d_found, ca, ceq = fc
                c = 15 - j

                def lsum(l, tot):
                    return tot + hist_v[pl.ds(l * HIST8 + c * LANES, LANES)]
                totals = lax.fori_loop(0, LANES, lsum, zeros_i)
                rev = jnp.flip(totals)
                cs = jnp.cumsum(rev)
                cond = (cs + acc) >= krem
                j0 = jnp.min(jnp.where(cond, lane_iota, jnp.int32(16)))
                excl = acc + jnp.sum(
                    jnp.where(lane_iota < j0, rev, jnp.int32(0)))
                eqv = jnp.sum(jnp.where(lane_iota == j0, rev, jnp.int32(0)))
                newly = jnp.logical_and(d_found < 0, j0 < 16)
                d_found = jnp.where(newly, c * 16 + 15 - j0, d_found)
                ca = jnp.where(newly, excl, ca)
                ceq = jnp.where(newly, eqv, ceq)
                acc = acc + jnp.sum(totals)
                return acc, d_found, ca, ceq
            _, d, ca, ceq = lax.fori_loop(
                0, 16, fb,
                (jnp.int32(0), jnp.int32(-1), jnp.int32(0), jnp.int32(0)))
            return d, ca, ceq

        def full_round(shift, pm, pb, krem, masked):
            zero_hist(LANES * HIST8)

            def hb(o, _):
                for s in range(4):
                    i = o * 4 + s
                    u = _to_ord(row_v[pl.ds(i * LANES, LANES)])
                    digit = ((u >> shift) & jnp.uint32(0xFF)).astype(jnp.int32)
                    if masked:
                        matc = (u & pm) == pb
                        plsc.addupdate_scatter(
                            hist_v, [lane_base8 + digit], ones_i, mask=matc)
                    else:
                        plsc.addupdate_scatter(
                            hist_v, [lane_base8 + digit], ones_i)
                return 0
            lax.fori_loop(0, nchunks // 4, hb, 0)

            d, ca, ceq = find8(krem)
            pb = pb | (d.astype(jnp.uint32) << shift)
            pm = pm | (jnp.uint32(0xFF) << shift)
            krem = krem - ca
            return pm, pb, krem, ceq

        def process_row(r):
            pltpu.sync_copy(x_hbm.at[r], row_v)

            pm, pb, krem, ceq = full_round(
                jnp.uint32(24), jnp.uint32(0), jnp.uint32(0),
                jnp.int32(TOPK), masked=False)

            def esc_body(rnd, carry):
                pm, pb, krem, ceq = carry

                def run(_):
                    shift = jnp.uint32(24) - jnp.uint32(8) * rnd.astype(
                        jnp.uint32)
                    return full_round(shift, pm, pb, krem, masked=True)

                return lax.cond(
                    (TOPK - krem) + ceq > CAP, run,
                    lambda _: (pm, pb, krem, ceq), 0)
            pm, pb, krem, ceq = lax.fori_loop(
                1, 4, esc_body, (pm, pb, krem, ceq))

            cnt_c = (TOPK - krem) + ceq
            allfull = pm == jnp.uint32(0xFFFFFFFF)

            def resolve_compact(_):
                pm_v = jnp.broadcast_to(pm, (LANES,))
                pb_v = jnp.broadcast_to(pb, (LANES,))

                def cb(o, ptr):
                    for s in range(4):
                        i = o * 4 + s
                        u = _to_ord(row_v[pl.ds(i * LANES, LANES)])
                        matc = (u & pm_v) >= pb_v
                        idxv = lane_iota + i * LANES
                        plsc.store_compressed(
                            cand_u.at[pl.ds(ptr, LANES)], u, mask=matc)
                        plsc.store_compressed(
                            cand_i.at[pl.ds(ptr, LANES)], idxv, mask=matc)
                        ptr = ptr + jnp.sum(matc.astype(jnp.int32))
                    return ptr
                lax.fori_loop(0, nchunks // 4, cb, jnp.int32(0))

                cchunks = (cnt_c + LANES - 1) // LANES
                cnt_v = jnp.broadcast_to(cnt_c, (LANES,))
                nbits = (jnp.uint32(32)
                         - jnp.uint32(8) * _popcount_bytes(pm))

                def cr_body(i, carry):
                    pm2, pb2, krem2 = carry
                    shift = nbits - jnp.uint32(4) - jnp.uint32(4) * i.astype(
                        jnp.uint32)
                    zero_hist(LANES * HIST4)

                    def chb(j, _):
                        u = cand_u[pl.ds(j * LANES, LANES)]
                        valid = (j * LANES + lane_iota) < cnt_v
                        matc = jnp.logical_and(valid, (u & pm2) == pb2)
                        digit = ((u >> shift)
                                 & jnp.uint32(0xF)).astype(jnp.int32)
                        plsc.addupdate_scatter(
                            hist_v, [lane_base4 + digit], ones_i, mask=matc)
                        return 0
                    lax.fori_loop(0, cchunks, chb, 0)

                    def lsum(l, tot):
                        return tot + hist_v[pl.ds(l * HIST4, LANES)]
                    totals = lax.fori_loop(0, LANES, lsum, zeros_i)
                    rev = jnp.flip(totals)
                    cs = jnp.cumsum(rev)
                    j0 = jnp.min(jnp.where(cs >= krem2, lane_iota,
                                           jnp.int32(16)))
                    excl = jnp.sum(jnp.where(lane_iota < j0, rev,
                                             jnp.int32(0)))
                    d = (15 - j0).astype(jnp.uint32)
                    pb2 = pb2 | (d << shift)
                    pm2 = pm2 | (jnp.uint32(0xF) << shift)
                    krem2 = krem2 - excl
                    return pm2, pb2, krem2

                _, t, m = lax.fori_loop(
                    0, nbits.astype(jnp.int32) // 4, cr_body, (pm, pb, krem))
                return t, m

            t, m = lax.cond(allfull, lambda _: (pb, krem), resolve_compact, 0)
            t_v = jnp.broadcast_to(t, (LANES,))
            m_v = jnp.broadcast_to(m, (LANES,))

            def emit_scatter(_):
                cchunks = (cnt_c + LANES - 1) // LANES
                cnt_v = jnp.broadcast_to(cnt_c, (LANES,))

                def sb(i, eqrun):
                    u = cand_u[pl.ds(i * LANES, LANES)]
                    idx = cand_i[pl.ds(i * LANES, LANES)]
                    valid = (i * LANES + lane_iota) < cnt_v
                    gt = jnp.logical_and(u > t_v, valid)
                    eq = jnp.logical_and(u == t_v, valid)
                    cs = jnp.cumsum(eq.astype(jnp.int32))
                    keep = jnp.logical_or(
                        gt, jnp.logical_and(eq, (cs + eqrun) <= m_v))
                    plsc.store_scatter(out_v, [idx], _from_ord(u), mask=keep)
                    return eqrun + jnp.max(cs)
                lax.fori_loop(0, cchunks, sb, jnp.int32(0))
                return 0

            def emit_scan(_):
                def ob(i, eqrun):
                    xv = row_v[pl.ds(i * LANES, LANES)]
                    u = _to_ord(xv)
                    gt = u > t_v
                    eq = u == t_v
                    cs = jnp.cumsum(eq.astype(jnp.int32))
                    keep = jnp.logical_or(
                        gt, jnp.logical_and(eq, (cs + eqrun) <= m_v))
                    out_v[pl.ds(i * LANES, LANES)] = jnp.where(
                        keep, xv, zeros_f)
                    return eqrun + jnp.max(cs)
                lax.fori_loop(0, nchunks, ob, jnp.int32(0))
                return 0

            lax.cond(allfull, emit_scan, emit_scatter, 0)
            pltpu.sync_copy(out_v, out_hbm.at[r])

            def restore_scatter(_):
                cchunks = (cnt_c + LANES - 1) // LANES
                cnt_v = jnp.broadcast_to(cnt_c, (LANES,))

                def rb(i, _):
                    idx = cand_i[pl.ds(i * LANES, LANES)]
                    valid = (i * LANES + lane_iota) < cnt_v
                    plsc.store_scatter(out_v, [idx], zeros_f, mask=valid)
                    return 0
                lax.fori_loop(0, cchunks, rb, 0)
                return 0

            def restore_all(_):
                def zb(i, _):
                    out_v[pl.ds(i * LANES, LANES)] = zeros_f
                    return 0
                lax.fori_loop(0, nchunks, zb, 0)
                return 0

            lax.cond(allfull, restore_all, restore_scatter, 0)

        def zout(i, _):
            out_v[pl.ds(i * LANES, LANES)] = zeros_f
            return 0
        lax.fori_loop(0, nchunks, zout, 0)

        for rr in range(rows_per_worker):
            process_row(wid * rows_per_worker + rr)

    return topk_kernel


def _popcount_bytes(pm):
    b0 = (pm >> jnp.uint32(24)) & jnp.uint32(1)
    b1 = (pm >> jnp.uint32(16)) & jnp.uint32(1)
    b2 = (pm >> jnp.uint32(8)) & jnp.uint32(1)
    b3 = pm & jnp.uint32(1)
    return b0 + b1 + b2 + b3


@jax.jit
def kernel(x):
    nrows, ncols = x.shape
    return _make_topk_kernel(nrows, ncols)(x)


# speedup vs baseline: 2.4211x; 1.3670x over previous
import functools

import jax
import jax.numpy as jnp
from jax import lax
from jax.experimental import pallas as pl
from jax.experimental.pallas import tpu as pltpu
from jax.experimental.pallas import tpu_sc as plsc

TOPK = 32
LANES = 16
NUM_CORES = 2
NUM_SUBCORES = 16
NUM_WORKERS = NUM_CORES * NUM_SUBCORES
HIST8 = 256
HIST4 = 16
CAP = 4096

def _to_ord(xv):
    b = lax.bitcast_convert_type(xv, jnp.int32)
    flip = (b >> 31) | jnp.int32(-2147483648)
    return lax.bitcast_convert_type(b ^ flip, jnp.uint32)


def _from_ord(u):
    ui = lax.bitcast_convert_type(u, jnp.int32)
    flip = ((~ui) >> 31) | jnp.int32(-2147483648)
    return lax.bitcast_convert_type(ui ^ flip, jnp.float32)


def _make_topk_kernel(nrows, ncols):
    rows_per_worker = nrows // NUM_WORKERS
    nchunks = ncols // LANES
    mesh = plsc.VectorSubcoreMesh(core_axis_name="c", subcore_axis_name="s")

    @functools.partial(
        pl.kernel,
        mesh=mesh,
        compiler_params=pltpu.CompilerParams(needs_layout_passes=False),
        out_type=jax.ShapeDtypeStruct((nrows, ncols), jnp.float32),
        scratch_types=[
            pltpu.VMEM((ncols,), jnp.float32),
            pltpu.VMEM((ncols,), jnp.float32),
            pltpu.VMEM((ncols,), jnp.float32),
            pltpu.VMEM((LANES * HIST8,), jnp.int32),
            pltpu.VMEM((CAP + LANES,), jnp.uint32),
            pltpu.VMEM((CAP + LANES,), jnp.int32),
            pltpu.SemaphoreType.DMA,
            pltpu.SemaphoreType.DMA,
        ],
    )
    def topk_kernel(x_hbm, out_hbm, row_a, row_b, out_v, hist_v, cand_u,
                    cand_i, sem_a, sem_b):
        wid = lax.axis_index("s") * NUM_CORES + lax.axis_index("c")
        lane_iota = lax.iota(jnp.int32, LANES)
        lane_base8 = lane_iota * HIST8
        lane_base4 = lane_iota * HIST4
        ones_i = jnp.ones((LANES,), jnp.int32)
        zeros_i = jnp.zeros((LANES,), jnp.int32)
        zeros_f = jnp.zeros((LANES,), jnp.float32)

        def find8(krem):
            def fb(j, fc):
                acc, d_found, ca, ceq = fc
                c = 15 - j

                def lsum(l, tot):
                    v = hist_v[pl.ds(l * HIST8 + c * LANES, LANES)]
                    hist_v[pl.ds(l * HIST8 + c * LANES, LANES)] = zeros_i
                    return tot + v
                totals = lax.fori_loop(0, LANES, lsum, zeros_i)
                rev = jnp.flip(totals)
                cs = jnp.cumsum(rev)
                cond = (cs + acc) >= krem
                j0 = jnp.min(jnp.where(cond, lane_iota, jnp.int32(16)))
                excl = acc + jnp.sum(
                    jnp.where(lane_iota < j0, rev, jnp.int32(0)))
                eqv = jnp.sum(jnp.where(lane_iota == j0, rev, jnp.int32(0)))
                newly = jnp.logical_and(d_found < 0, j0 < 16)
                d_found = jnp.where(newly, c * 16 + 15 - j0, d_found)
                ca = jnp.where(newly, excl, ca)
                ceq = jnp.where(newly, eqv, ceq)
                acc = acc + jnp.sum(totals)
                return acc, d_found, ca, ceq
            _, d, ca, ceq = lax.fori_loop(
                0, 16, fb,
                (jnp.int32(0), jnp.int32(-1), jnp.int32(0), jnp.int32(0)))
            return d, ca, ceq

        def full_round(row_v, shift, pm, pb, krem, masked):
            def hb(o, _):
                for s in range(8):
                    i = o * 8 + s
                    u = _to_ord(row_v[pl.ds(i * LANES, LANES)])
                    digit = ((u >> shift) & jnp.uint32(0xFF)).astype(jnp.int32)
                    if masked:
                        matc = (u & pm) == pb
                        plsc.addupdate_scatter(
                            hist_v, [lane_base8 + digit], ones_i, mask=matc)
                    else:
                        plsc.addupdate_scatter(
                            hist_v, [lane_base8 + digit], ones_i)
                return 0
            lax.fori_loop(0, nchunks // 8, hb, 0)

            d, ca, ceq = find8(krem)
            pb = pb | (d.astype(jnp.uint32) << shift)
            pm = pm | (jnp.uint32(0xFF) << shift)
            krem = krem - ca
            return pm, pb, krem, ceq

        def process_row(r, row_v):
            pm, pb, krem, ceq = full_round(
                row_v, jnp.uint32(24), jnp.uint32(0), jnp.uint32(0),
                jnp.int32(TOPK), masked=False)

            def esc_body(rnd, carry):
                pm, pb, krem, ceq = carry

                def run(_):
                    shift = jnp.uint32(24) - jnp.uint32(8) * rnd.astype(
                        jnp.uint32)
                    return full_round(row_v, shift, pm, pb, krem, masked=True)

                return lax.cond(
                    (TOPK - krem) + ceq > CAP, run,
                    lambda _: (pm, pb, krem, ceq), 0)
            pm, pb, krem, ceq = lax.fori_loop(
                1, 4, esc_body, (pm, pb, krem, ceq))

            cnt_c = (TOPK - krem) + ceq
            allfull = pm == jnp.uint32(0xFFFFFFFF)

            def resolve_compact(_):
                pm_v = jnp.broadcast_to(pm, (LANES,))
                pb_v = jnp.broadcast_to(pb, (LANES,))

                def cb(o, ptr):
                    us, masks, cnts = [], [], []
                    for s in range(8):
                        i = o * 8 + s
                        u = _to_ord(row_v[pl.ds(i * LANES, LANES)])
                        matc = (u & pm_v) >= pb_v
                        us.append(u)
                        masks.append(matc)
                        cnts.append(jnp.sum(matc.astype(jnp.int32)))
                    offs = [ptr]
                    for s in range(8):
                        offs.append(offs[s] + cnts[s])
                    for s in range(8):
                        i = o * 8 + s
                        plsc.store_compressed(
                            cand_u.at[pl.ds(offs[s], LANES)], us[s],
                            mask=masks[s])
                        plsc.store_compressed(
                            cand_i.at[pl.ds(offs[s], LANES)],
                            lane_iota + i * LANES, mask=masks[s])
                    return offs[8]
                lax.fori_loop(0, nchunks // 8, cb, jnp.int32(0))

                cchunks = (cnt_c + LANES - 1) // LANES
                cnt_v = jnp.broadcast_to(cnt_c, (LANES,))
                nbits = (jnp.uint32(32)
                         - jnp.uint32(8) * _popcount_bytes(pm))

                def cr_body(i, carry):
                    pm2, pb2, krem2 = carry
                    shift = nbits - jnp.uint32(4) - jnp.uint32(4) * i.astype(
                        jnp.uint32)

                    def chb(j, _):
                        u = cand_u[pl.ds(j * LANES, LANES)]
                        valid = (j * LANES + lane_iota) < cnt_v
                        matc = jnp.logical_and(valid, (u & pm2) == pb2)
                        digit = ((u >> shift)
                                 & jnp.uint32(0xF)).astype(jnp.int32)
                        plsc.addupdate_scatter(
                            hist_v, [lane_base4 + digit], ones_i, mask=matc)
                        return 0
                    lax.fori_loop(0, cchunks, chb, 0)

                    def lsum(l, tot):
                        v = hist_v[pl.ds(l * HIST4, LANES)]
                        hist_v[pl.ds(l * HIST4, LANES)] = zeros_i
                        return tot + v
                    totals = lax.fori_loop(0, LANES, lsum, zeros_i)
                    rev = jnp.flip(totals)
                    cs = jnp.cumsum(rev)
                    j0 = jnp.min(jnp.where(cs >= krem2, lane_iota,
                                           jnp.int32(16)))
                    excl = jnp.sum(jnp.where(lane_iota < j0, rev,
                                             jnp.int32(0)))
                    d = (15 - j0).astype(jnp.uint32)
                    pb2 = pb2 | (d << shift)
                    pm2 = pm2 | (jnp.uint32(0xF) << shift)
                    krem2 = krem2 - excl
                    return pm2, pb2, krem2

                _, t, m = lax.fori_loop(
                    0, nbits.astype(jnp.int32) // 4, cr_body, (pm, pb, krem))
                return t, m

            t, m = lax.cond(allfull, lambda _: (pb, krem), resolve_compact, 0)
            t_v = jnp.broadcast_to(t, (LANES,))
            m_v = jnp.broadcast_to(m, (LANES,))

            def emit_scatter(_):
                cchunks = (cnt_c + LANES - 1) // LANES
                cnt_v = jnp.broadcast_to(cnt_c, (LANES,))

                def sb(i, eqrun):
                    u = cand_u[pl.ds(i * LANES, LANES)]
                    idx = cand_i[pl.ds(i * LANES, LANES)]
                    valid = (i * LANES + lane_iota) < cnt_v
                    gt = jnp.logical_and(u > t_v, valid)
                    eq = jnp.logical_and(u == t_v, valid)
                    cs = jnp.cumsum(eq.astype(jnp.int32))
                    keep = jnp.logical_or(
                        gt, jnp.logical_and(eq, (cs + eqrun) <= m_v))
                    plsc.store_scatter(out_v, [idx], _from_ord(u), mask=keep)
                    return eqrun + jnp.max(cs)
                lax.fori_loop(0, cchunks, sb, jnp.int32(0))
                return 0

            def emit_scan(_):
                def ob(i, eqrun):
                    xv = row_v[pl.ds(i * LANES, LANES)]
                    u = _to_ord(xv)
                    gt = u > t_v
                    eq = u == t_v
                    cs = jnp.cumsum(eq.astype(jnp.int32))
                    keep = jnp.logical_or(
                        gt, jnp.logical_and(eq, (cs + eqrun) <= m_v))
                    out_v[pl.ds(i * LANES, LANES)] = jnp.where(
                        keep, xv, zeros_f)
                    return eqrun + jnp.max(cs)
                lax.fori_loop(0, nchunks, ob, jnp.int32(0))
                return 0

            lax.cond(allfull, emit_scan, emit_scatter, 0)
            pltpu.sync_copy(out_v, out_hbm.at[r])

            def restore_scatter(_):
                cchunks = (cnt_c + LANES - 1) // LANES
                cnt_v = jnp.broadcast_to(cnt_c, (LANES,))

                def rb(i, _):
                    idx = cand_i[pl.ds(i * LANES, LANES)]
                    valid = (i * LANES + lane_iota) < cnt_v
                    plsc.store_scatter(out_v, [idx], zeros_f, mask=valid)
                    return 0
                lax.fori_loop(0, cchunks, rb, 0)
                return 0

            def restore_all(_):
                def zb(i, _):
                    out_v[pl.ds(i * LANES, LANES)] = zeros_f
                    return 0
                lax.fori_loop(0, nchunks, zb, 0)
                return 0

            lax.cond(allfull, restore_all, restore_scatter, 0)

        r0 = wid * rows_per_worker
        cp_a = pltpu.async_copy(x_hbm.at[r0], row_a, sem_a)
        cp_b = pltpu.async_copy(x_hbm.at[r0 + 1], row_b, sem_b)

        def zout(i, _):
            out_v[pl.ds(i * LANES, LANES)] = zeros_f
            return 0
        lax.fori_loop(0, nchunks, zout, 0)

        def zhist(i, _):
            hist_v[pl.ds(i * LANES, LANES)] = zeros_i
            return 0
        lax.fori_loop(0, (LANES * HIST8) // LANES, zhist, 0)

        cp_a.wait()
        process_row(r0, row_a)
        cp_b.wait()
        process_row(r0 + 1, row_b)

    return topk_kernel


def _popcount_bytes(pm):
    b0 = (pm >> jnp.uint32(24)) & jnp.uint32(1)
    b1 = (pm >> jnp.uint32(16)) & jnp.uint32(1)
    b2 = (pm >> jnp.uint32(8)) & jnp.uint32(1)
    b3 = pm & jnp.uint32(1)
    return b0 + b1 + b2 + b3


@jax.jit
def kernel(x):
    nrows, ncols = x.shape
    return _make_topk_kernel(nrows, ncols)(x)


# speedup vs baseline: 2.6735x; 1.1043x over previous
import functools

import jax
import jax.numpy as jnp
from jax import lax
from jax.experimental import pallas as pl
from jax.experimental.pallas import tpu as pltpu
from jax.experimental.pallas import tpu_sc as plsc

TOPK = 32
LANES = 16
NUM_CORES = 2
NUM_SUBCORES = 16
NUM_WORKERS = NUM_CORES * NUM_SUBCORES
HIST8 = 256
HIST4 = 16
CAP = 4096

def _to_ord(xv):
    b = lax.bitcast_convert_type(xv, jnp.int32)
    flip = (b >> 31) | jnp.int32(-2147483648)
    return lax.bitcast_convert_type(b ^ flip, jnp.uint32)


def _from_ord(u):
    ui = lax.bitcast_convert_type(u, jnp.int32)
    flip = ((~ui) >> 31) | jnp.int32(-2147483648)
    return lax.bitcast_convert_type(ui ^ flip, jnp.float32)


def _make_topk_kernel(nrows, ncols):
    rows_per_worker = nrows // NUM_WORKERS
    nchunks = ncols // LANES
    mesh = plsc.VectorSubcoreMesh(core_axis_name="c", subcore_axis_name="s")

    @functools.partial(
        pl.kernel,
        mesh=mesh,
        compiler_params=pltpu.CompilerParams(needs_layout_passes=False),
        out_type=jax.ShapeDtypeStruct((nrows, ncols), jnp.float32),
        scratch_types=[
            pltpu.VMEM((ncols,), jnp.float32),
            pltpu.VMEM((ncols,), jnp.float32),
            pltpu.VMEM((ncols,), jnp.float32),
            pltpu.VMEM((LANES * HIST8,), jnp.int32),
            pltpu.VMEM((CAP + LANES,), jnp.uint32),
            pltpu.VMEM((CAP + LANES,), jnp.int32),
            pltpu.SemaphoreType.DMA,
            pltpu.SemaphoreType.DMA,
        ],
    )
    def topk_kernel(x_hbm, out_hbm, row_a, row_b, out_v, hist_v, cand_u,
                    cand_i, sem_a, sem_b):
        wid = lax.axis_index("s") * NUM_CORES + lax.axis_index("c")
        lane_iota = lax.iota(jnp.int32, LANES)
        ones_i = jnp.ones((LANES,), jnp.int32)
        zeros_i = jnp.zeros((LANES,), jnp.int32)
        zeros_f = jnp.zeros((LANES,), jnp.float32)

        def find_top(krem, nbuckets):
            def wcond(carry):
                _, cum, _ = carry
                return cum < krem

            def wbody(carry):
                c, cum, _ = carry
                v = hist_v[pl.ds(c * LANES, LANES)]
                hist_v[pl.ds(c * LANES, LANES)] = zeros_i
                return c - 1, cum + jnp.sum(v), cum
            c, cum, prev = lax.while_loop(
                wcond, wbody,
                (jnp.int32(nbuckets - 1), jnp.int32(0), jnp.int32(0)))
            d = c + 1

            def zb(b, _):
                hist_v[pl.ds(b * LANES, LANES)] = zeros_i
                return 0
            lax.fori_loop(0, d, zb, 0)
            return d, prev, cum - prev

        def full_round(row_v, shift, pm, pb, krem, masked):
            def hb(o, _):
                for s in range(8):
                    i = o * 8 + s
                    u = _to_ord(row_v[pl.ds(i * LANES, LANES)])
                    digit = ((u >> shift) & jnp.uint32(0xFF)).astype(jnp.int32)
                    idx = digit * LANES + lane_iota
                    if masked:
                        matc = (u & pm) == pb
                        plsc.addupdate_scatter(hist_v, [idx], ones_i,
                                               mask=matc)
                    else:
                        plsc.addupdate_scatter(hist_v, [idx], ones_i)
                return 0
            lax.fori_loop(0, nchunks // 8, hb, 0)

            d, ca, ceq = find_top(krem, HIST8)
            pb = pb | (d.astype(jnp.uint32) << shift)
            pm = pm | (jnp.uint32(0xFF) << shift)
            krem = krem - ca
            return pm, pb, krem, ceq

        def process_row(r, row_v):
            pm, pb, krem, ceq = full_round(
                row_v, jnp.uint32(24), jnp.uint32(0), jnp.uint32(0),
                jnp.int32(TOPK), masked=False)

            def esc_body(rnd, carry):
                pm, pb, krem, ceq = carry

                def run(_):
                    shift = jnp.uint32(24) - jnp.uint32(8) * rnd.astype(
                        jnp.uint32)
                    return full_round(row_v, shift, pm, pb, krem, masked=True)

                return lax.cond(
                    (TOPK - krem) + ceq > CAP, run,
                    lambda _: (pm, pb, krem, ceq), 0)
            pm, pb, krem, ceq = lax.fori_loop(
                1, 4, esc_body, (pm, pb, krem, ceq))

            cnt_c = (TOPK - krem) + ceq
            allfull = pm == jnp.uint32(0xFFFFFFFF)

            def resolve_compact(_):
                pm_v = jnp.broadcast_to(pm, (LANES,))
                pb_v = jnp.broadcast_to(pb, (LANES,))

                def cb(o, ptr):
                    us, masks, cnts = [], [], []
                    for s in range(8):
                        i = o * 8 + s
                        u = _to_ord(row_v[pl.ds(i * LANES, LANES)])
                        matc = (u & pm_v) >= pb_v
                        us.append(u)
                        masks.append(matc)
                        cnts.append(jnp.sum(matc.astype(jnp.int32)))
                    offs = [ptr]
                    for s in range(8):
                        offs.append(offs[s] + cnts[s])
                    for s in range(8):
                        i = o * 8 + s
                        plsc.store_compressed(
                            cand_u.at[pl.ds(offs[s], LANES)], us[s],
                            mask=masks[s])
                        plsc.store_compressed(
                            cand_i.at[pl.ds(offs[s], LANES)],
                            lane_iota + i * LANES, mask=masks[s])
                    return offs[8]
                lax.fori_loop(0, nchunks // 8, cb, jnp.int32(0))

                cchunks = (cnt_c + LANES - 1) // LANES
                cnt_v = jnp.broadcast_to(cnt_c, (LANES,))
                nbits = (jnp.uint32(32)
                         - jnp.uint32(8) * _popcount_bytes(pm))

                def cr_body(i, carry):
                    pm2, pb2, krem2 = carry
                    shift = nbits - jnp.uint32(4) - jnp.uint32(4) * i.astype(
                        jnp.uint32)

                    def chb(j, _):
                        u = cand_u[pl.ds(j * LANES, LANES)]
                        valid = (j * LANES + lane_iota) < cnt_v
                        matc = jnp.logical_and(valid, (u & pm2) == pb2)
                        digit = ((u >> shift)
                                 & jnp.uint32(0xF)).astype(jnp.int32)
                        plsc.addupdate_scatter(
                            hist_v, [digit * LANES + lane_iota], ones_i,
                            mask=matc)
                        return 0
                    lax.fori_loop(0, cchunks, chb, 0)

                    d, excl, _ = find_top(krem2, HIST4)
                    pb2 = pb2 | (d.astype(jnp.uint32) << shift)
                    pm2 = pm2 | (jnp.uint32(0xF) << shift)
                    krem2 = krem2 - excl
                    return pm2, pb2, krem2

                _, t, m = lax.fori_loop(
                    0, nbits.astype(jnp.int32) // 4, cr_body, (pm, pb, krem))
                return t, m

            t, m = lax.cond(allfull, lambda _: (pb, krem), resolve_compact, 0)
            t_v = jnp.broadcast_to(t, (LANES,))
            m_v = jnp.broadcast_to(m, (LANES,))

            def emit_scatter(_):
                cchunks = (cnt_c + LANES - 1) // LANES
                cnt_v = jnp.broadcast_to(cnt_c, (LANES,))

                def sb(i, eqrun):
                    u = cand_u[pl.ds(i * LANES, LANES)]
                    idx = cand_i[pl.ds(i * LANES, LANES)]
                    valid = (i * LANES + lane_iota) < cnt_v
                    gt = jnp.logical_and(u > t_v, valid)
                    eq = jnp.logical_and(u == t_v, valid)
                    cs = jnp.cumsum(eq.astype(jnp.int32))
                    keep = jnp.logical_or(
                        gt, jnp.logical_and(eq, (cs + eqrun) <= m_v))
                    plsc.store_scatter(out_v, [idx], _from_ord(u), mask=keep)
                    return eqrun + jnp.max(cs)
                lax.fori_loop(0, cchunks, sb, jnp.int32(0))
                return 0

            def emit_scan(_):
                def ob(i, eqrun):
                    xv = row_v[pl.ds(i * LANES, LANES)]
                    u = _to_ord(xv)
                    gt = u > t_v
                    eq = u == t_v
                    cs = jnp.cumsum(eq.astype(jnp.int32))
                    keep = jnp.logical_or(
                        gt, jnp.logical_and(eq, (cs + eqrun) <= m_v))
                    out_v[pl.ds(i * LANES, LANES)] = jnp.where(
                        keep, xv, zeros_f)
                    return eqrun + jnp.max(cs)
                lax.fori_loop(0, nchunks, ob, jnp.int32(0))
                return 0

            lax.cond(allfull, emit_scan, emit_scatter, 0)
            pltpu.sync_copy(out_v, out_hbm.at[r])

            def restore_scatter(_):
                cchunks = (cnt_c + LANES - 1) // LANES
                cnt_v = jnp.broadcast_to(cnt_c, (LANES,))

                def rb(i, _):
                    idx = cand_i[pl.ds(i * LANES, LANES)]
                    valid = (i * LANES + lane_iota) < cnt_v
                    plsc.store_scatter(out_v, [idx], zeros_f, mask=valid)
                    return 0
                lax.fori_loop(0, cchunks, rb, 0)
                return 0

            def restore_all(_):
                def zb(i, _):
                    out_v[pl.ds(i * LANES, LANES)] = zeros_f
                    return 0
                lax.fori_loop(0, nchunks, zb, 0)
                return 0

            lax.cond(allfull, restore_all, restore_scatter, 0)

        r0 = wid * rows_per_worker
        cp_a = pltpu.async_copy(x_hbm.at[r0], row_a, sem_a)
        cp_b = pltpu.async_copy(x_hbm.at[r0 + 1], row_b, sem_b)

        def zout(i, _):
            out_v[pl.ds(i * LANES, LANES)] = zeros_f
            return 0
        lax.fori_loop(0, nchunks, zout, 0)

        def zhist(i, _):
            hist_v[pl.ds(i * LANES, LANES)] = zeros_i
            return 0
        lax.fori_loop(0, (LANES * HIST8) // LANES, zhist, 0)

        cp_a.wait()
        process_row(r0, row_a)
        cp_b.wait()
        process_row(r0 + 1, row_b)

    return topk_kernel


def _popcount_bytes(pm):
    b0 = (pm >> jnp.uint32(24)) & jnp.uint32(1)
    b1 = (pm >> jnp.uint32(16)) & jnp.uint32(1)
    b2 = (pm >> jnp.uint32(8)) & jnp.uint32(1)
    b3 = pm & jnp.uint32(1)
    return b0 + b1 + b2 + b3


@jax.jit
def kernel(x):
    nrows, ncols = x.shape
    return _make_topk_kernel(nrows, ncols)(x)


# speedup vs baseline: 2.6745x; 1.0003x over previous
import functools

import jax
import jax.numpy as jnp
from jax import lax
from jax.experimental import pallas as pl
from jax.experimental.pallas import tpu as pltpu
from jax.experimental.pallas import tpu_sc as plsc

TOPK = 32
LANES = 16
NUM_CORES = 2
NUM_SUBCORES = 16
NUM_WORKERS = NUM_CORES * NUM_SUBCORES
HIST8 = 256
HIST4 = 16
CAP = 4096

def _to_ord(xv):
    b = lax.bitcast_convert_type(xv, jnp.int32)
    flip = (b >> 31) | jnp.int32(-2147483648)
    return lax.bitcast_convert_type(b ^ flip, jnp.uint32)


def _from_ord(u):
    ui = lax.bitcast_convert_type(u, jnp.int32)
    flip = ((~ui) >> 31) | jnp.int32(-2147483648)
    return lax.bitcast_convert_type(ui ^ flip, jnp.float32)


def _make_topk_kernel(nrows, ncols):
    rows_per_worker = nrows // NUM_WORKERS
    nchunks = ncols // LANES
    mesh = plsc.VectorSubcoreMesh(core_axis_name="c", subcore_axis_name="s")

    @functools.partial(
        pl.kernel,
        mesh=mesh,
        compiler_params=pltpu.CompilerParams(needs_layout_passes=False),
        out_type=jax.ShapeDtypeStruct((nrows, ncols), jnp.float32),
        scratch_types=[
            pltpu.VMEM((ncols,), jnp.float32),
            pltpu.VMEM((ncols,), jnp.float32),
            pltpu.VMEM((ncols,), jnp.float32),
            pltpu.VMEM((LANES * HIST8,), jnp.int32),
            pltpu.VMEM((CAP + LANES,), jnp.uint32),
            pltpu.VMEM((CAP + LANES,), jnp.int32),
            pltpu.SemaphoreType.DMA,
            pltpu.SemaphoreType.DMA,
        ],
    )
    def topk_kernel(x_hbm, out_hbm, row_a, row_b, out_v, hist_v, cand_u,
                    cand_i, sem_a, sem_b):
        wid = lax.axis_index("s") * NUM_CORES + lax.axis_index("c")
        lane_iota = lax.iota(jnp.int32, LANES)
        ones_i = jnp.ones((LANES,), jnp.int32)
        zeros_i = jnp.zeros((LANES,), jnp.int32)
        zeros_f = jnp.zeros((LANES,), jnp.float32)

        def find_top(krem, nbuckets):
            def wcond(carry):
                _, cum, _ = carry
                return cum < krem

            def wbody(carry):
                c, cum, _ = carry
                v = hist_v[pl.ds(c * LANES, LANES)]
                hist_v[pl.ds(c * LANES, LANES)] = zeros_i
                return c - 1, cum + jnp.sum(v), cum
            c, cum, prev = lax.while_loop(
                wcond, wbody,
                (jnp.int32(nbuckets - 1), jnp.int32(0), jnp.int32(0)))
            d = c + 1

            def zb(b, _):
                hist_v[pl.ds(b * LANES, LANES)] = zeros_i
                return 0
            lax.fori_loop(0, d, zb, 0)
            return d, prev, cum - prev

        def full_round(row_v, shift, pm, pb, krem, masked):
            def hb(o, _):
                for s in range(8):
                    i = o * 8 + s
                    u = _to_ord(row_v[pl.ds(i * LANES, LANES)])
                    digit = ((u >> shift) & jnp.uint32(0xFF)).astype(jnp.int32)
                    idx = digit * LANES + lane_iota
                    if masked:
                        matc = (u & pm) == pb
                        plsc.addupdate_scatter(hist_v, [idx], ones_i,
                                               mask=matc)
                    else:
                        plsc.addupdate_scatter(hist_v, [idx], ones_i)
                return 0
            lax.fori_loop(0, nchunks // 8, hb, 0)

            d, ca, ceq = find_top(krem, HIST8)
            pb = pb | (d.astype(jnp.uint32) << shift)
            pm = pm | (jnp.uint32(0xFF) << shift)
            krem = krem - ca
            return pm, pb, krem, ceq

        def process_row(r, row_v):
            with jax.named_scope("hist0"):
                pm, pb, krem, ceq = full_round(
                    row_v, jnp.uint32(24), jnp.uint32(0), jnp.uint32(0),
                    jnp.int32(TOPK), masked=False)

            def esc_body(rnd, carry):
                pm, pb, krem, ceq = carry

                def run(_):
                    shift = jnp.uint32(24) - jnp.uint32(8) * rnd.astype(
                        jnp.uint32)
                    return full_round(row_v, shift, pm, pb, krem, masked=True)

                return lax.cond(
                    (TOPK - krem) + ceq > CAP, run,
                    lambda _: (pm, pb, krem, ceq), 0)
            pm, pb, krem, ceq = lax.fori_loop(
                1, 4, esc_body, (pm, pb, krem, ceq))

            cnt_c = (TOPK - krem) + ceq
            allfull = pm == jnp.uint32(0xFFFFFFFF)

            def resolve_compact(_):
                pm_v = jnp.broadcast_to(pm, (LANES,))
                pb_v = jnp.broadcast_to(pb, (LANES,))

                def cb(o, ptr):
                    us, masks, cnts = [], [], []
                    for s in range(8):
                        i = o * 8 + s
                        u = _to_ord(row_v[pl.ds(i * LANES, LANES)])
                        matc = (u & pm_v) >= pb_v
                        us.append(u)
                        masks.append(matc)
                        cnts.append(jnp.sum(matc.astype(jnp.int32)))
                    offs = [ptr]
                    for s in range(8):
                        offs.append(offs[s] + cnts[s])
                    for s in range(8):
                        i = o * 8 + s
                        plsc.store_compressed(
                            cand_u.at[pl.ds(offs[s], LANES)], us[s],
                            mask=masks[s])
                        plsc.store_compressed(
                            cand_i.at[pl.ds(offs[s], LANES)],
                            lane_iota + i * LANES, mask=masks[s])
                    return offs[8]
                with jax.named_scope("compact"):
                    lax.fori_loop(0, nchunks // 8, cb, jnp.int32(0))

                cchunks = (cnt_c + LANES - 1) // LANES
                cnt_v = jnp.broadcast_to(cnt_c, (LANES,))
                nbits = (jnp.uint32(32)
                         - jnp.uint32(8) * _popcount_bytes(pm))

                def cr_body(i, carry):
                    pm2, pb2, krem2 = carry
                    shift = nbits - jnp.uint32(4) - jnp.uint32(4) * i.astype(
                        jnp.uint32)

                    def chb(j, _):
                        u = cand_u[pl.ds(j * LANES, LANES)]
                        valid = (j * LANES + lane_iota) < cnt_v
                        matc = jnp.logical_and(valid, (u & pm2) == pb2)
                        digit = ((u >> shift)
                                 & jnp.uint32(0xF)).astype(jnp.int32)
                        plsc.addupdate_scatter(
                            hist_v, [digit * LANES + lane_iota], ones_i,
                            mask=matc)
                        return 0
                    lax.fori_loop(0, cchunks, chb, 0)

                    d, excl, _ = find_top(krem2, HIST4)
                    pb2 = pb2 | (d.astype(jnp.uint32) << shift)
                    pm2 = pm2 | (jnp.uint32(0xF) << shift)
                    krem2 = krem2 - excl
                    return pm2, pb2, krem2

                with jax.named_scope("crounds"):
                    _, t, m = lax.fori_loop(
                        0, nbits.astype(jnp.int32) // 4, cr_body,
                        (pm, pb, krem))
                return t, m

            t, m = lax.cond(allfull, lambda _: (pb, krem), resolve_compact, 0)
            t_v = jnp.broadcast_to(t, (LANES,))
            m_v = jnp.broadcast_to(m, (LANES,))

            def emit_scatter(_):
                cchunks = (cnt_c + LANES - 1) // LANES
                cnt_v = jnp.broadcast_to(cnt_c, (LANES,))

                def sb(i, eqrun):
                    u = cand_u[pl.ds(i * LANES, LANES)]
                    idx = cand_i[pl.ds(i * LANES, LANES)]
                    valid = (i * LANES + lane_iota) < cnt_v
                    gt = jnp.logical_and(u > t_v, valid)
                    eq = jnp.logical_and(u == t_v, valid)
                    cs = jnp.cumsum(eq.astype(jnp.int32))
                    keep = jnp.logical_or(
                        gt, jnp.logical_and(eq, (cs + eqrun) <= m_v))
                    plsc.store_scatter(out_v, [idx], _from_ord(u), mask=keep)
                    return eqrun + jnp.max(cs)
                lax.fori_loop(0, cchunks, sb, jnp.int32(0))
                return 0

            def emit_scan(_):
                def ob(i, eqrun):
                    xv = row_v[pl.ds(i * LANES, LANES)]
                    u = _to_ord(xv)
                    gt = u > t_v
                    eq = u == t_v
                    cs = jnp.cumsum(eq.astype(jnp.int32))
                    keep = jnp.logical_or(
                        gt, jnp.logical_and(eq, (cs + eqrun) <= m_v))
                    out_v[pl.ds(i * LANES, LANES)] = jnp.where(
                        keep, xv, zeros_f)
                    return eqrun + jnp.max(cs)
                lax.fori_loop(0, nchunks, ob, jnp.int32(0))
                return 0

            with jax.named_scope("emit"):
                lax.cond(allfull, emit_scan, emit_scatter, 0)
            with jax.named_scope("outdma"):
                pltpu.sync_copy(out_v, out_hbm.at[r])

            def restore_scatter(_):
                cchunks = (cnt_c + LANES - 1) // LANES
                cnt_v = jnp.broadcast_to(cnt_c, (LANES,))

                def rb(i, _):
                    idx = cand_i[pl.ds(i * LANES, LANES)]
                    valid = (i * LANES + lane_iota) < cnt_v
                    plsc.store_scatter(out_v, [idx], zeros_f, mask=valid)
                    return 0
                lax.fori_loop(0, cchunks, rb, 0)
                return 0

            def restore_all(_):
                def zb(i, _):
                    out_v[pl.ds(i * LANES, LANES)] = zeros_f
                    return 0
                lax.fori_loop(0, nchunks, zb, 0)
                return 0

            with jax.named_scope("restore"):
                lax.cond(allfull, restore_all, restore_scatter, 0)

        r0 = wid * rows_per_worker
        cp_a = pltpu.async_copy(x_hbm.at[r0], row_a, sem_a)
        cp_b = pltpu.async_copy(x_hbm.at[r0 + 1], row_b, sem_b)

        with jax.named_scope("init"):
            def zout(i, _):
                out_v[pl.ds(i * LANES, LANES)] = zeros_f
                return 0
            lax.fori_loop(0, nchunks, zout, 0)

            def zhist(i, _):
                hist_v[pl.ds(i * LANES, LANES)] = zeros_i
                return 0
            lax.fori_loop(0, (LANES * HIST8) // LANES, zhist, 0)

        with jax.named_scope("indma_a"):
            cp_a.wait()
        process_row(r0, row_a)
        with jax.named_scope("indma_b"):
            cp_b.wait()
        process_row(r0 + 1, row_b)

    return topk_kernel


def _popcount_bytes(pm):
    b0 = (pm >> jnp.uint32(24)) & jnp.uint32(1)
    b1 = (pm >> jnp.uint32(16)) & jnp.uint32(1)
    b2 = (pm >> jnp.uint32(8)) & jnp.uint32(1)
    b3 = pm & jnp.uint32(1)
    return b0 + b1 + b2 + b3


@jax.jit
def kernel(x):
    nrows, ncols = x.shape
    return _make_topk_kernel(nrows, ncols)(x)


# speedup vs baseline: 4.3178x; 1.6144x over previous
import functools

import jax
import jax.numpy as jnp
from jax import lax
from jax.experimental import pallas as pl
from jax.experimental.pallas import tpu as pltpu
from jax.experimental.pallas import tpu_sc as plsc

TOPK = 32
LANES = 16
NUM_CORES = 2
NUM_SUBCORES = 16
NUM_WORKERS = NUM_CORES * NUM_SUBCORES
HIST8 = 256
HIST4 = 16
CAP = 4096


def _to_ord(xv):
    b = lax.bitcast_convert_type(xv, jnp.int32)
    flip = (b >> 31) | jnp.int32(-2147483648)
    return lax.bitcast_convert_type(b ^ flip, jnp.uint32)


def _popcount_bytes(pm):
    b0 = (pm >> jnp.uint32(24)) & jnp.uint32(1)
    b1 = (pm >> jnp.uint32(16)) & jnp.uint32(1)
    b2 = (pm >> jnp.uint32(8)) & jnp.uint32(1)
    b3 = pm & jnp.uint32(1)
    return b0 + b1 + b2 + b3


def _make_topk_kernel(nrows, ncols):
    rows_per_worker = nrows // NUM_WORKERS
    assert rows_per_worker == 2 and ncols % (8 * LANES) == 0
    nchunks = ncols // LANES
    mesh = plsc.VectorSubcoreMesh(core_axis_name="c", subcore_axis_name="s")

    @functools.partial(
        pl.kernel,
        mesh=mesh,
        compiler_params=pltpu.CompilerParams(needs_layout_passes=False),
        out_type=jax.ShapeDtypeStruct((nrows, ncols), jnp.float32),
        scratch_types=[
            pltpu.VMEM((ncols,), jnp.float32),
            pltpu.VMEM((ncols,), jnp.float32),
            pltpu.VMEM((ncols,), jnp.float32),
            pltpu.VMEM((LANES * HIST8,), jnp.int32),
            pltpu.VMEM((CAP + 144,), jnp.int32),
            pltpu.SemaphoreType.DMA,
            pltpu.SemaphoreType.DMA,
        ],
    )
    def topk_kernel(x_hbm, out_hbm, row_a, row_b, out_v, hist_v, cand_i,
                    sem_a, sem_b):
        wid = lax.axis_index("s") * NUM_CORES + lax.axis_index("c")
        lane_iota = lax.iota(jnp.int32, LANES)
        ones_i = jnp.ones((LANES,), jnp.int32)
        zeros_i = jnp.zeros((LANES,), jnp.int32)
        zeros_f = jnp.zeros((LANES,), jnp.float32)

        def top2_scan(row_v):
            zu = jnp.zeros((LANES,), jnp.uint32)

            def body(o, carry):
                m1s = list(carry[:8])
                m2s = list(carry[8:])
                for s in range(8):
                    u = _to_ord(row_v[pl.ds((o * 8 + s) * LANES, LANES)])
                    m2s[s] = jnp.maximum(m2s[s], jnp.minimum(m1s[s], u))
                    m1s[s] = jnp.maximum(m1s[s], u)
                return tuple(m1s + m2s)
            carry = lax.fori_loop(0, nchunks // 8, body, (zu,) * 16)

            pairs = list(zip(carry[:8], carry[8:]))
            while len(pairs) > 1:
                nxt = []
                for (a1, a2), (b1, b2) in zip(pairs[::2], pairs[1::2]):
                    hi = jnp.maximum(a1, b1)
                    lo = jnp.maximum(jnp.minimum(a1, b1),
                                     jnp.maximum(a2, b2))
                    nxt.append((hi, lo))
                pairs = nxt
            _, m2 = pairs[0]
            return jnp.min(m2)

        def compact(row_v, thresh):
            th_v = jnp.broadcast_to(thresh, (LANES,))

            def cb(o, ptr):
                base = jnp.minimum(ptr, jnp.int32(CAP))
                masks, cnts = [], []
                for s in range(8):
                    u = _to_ord(row_v[pl.ds((o * 8 + s) * LANES, LANES)])
                    matc = u >= th_v
                    masks.append(matc)
                    cnts.append(jnp.sum(matc.astype(jnp.int32)))
                offs = [base]
                for s in range(8):
                    offs.append(offs[s] + cnts[s])
                for s in range(8):
                    plsc.store_compressed(
                        cand_i.at[pl.ds(offs[s], LANES)],
                        lane_iota + (o * 8 + s) * LANES, mask=masks[s])
                return ptr + (offs[8] - base)
            return lax.fori_loop(0, nchunks // 8, cb, jnp.int32(0))

        def find_top(krem, nbuckets):
            def wcond(carry):
                _, cum, _ = carry
                return cum < krem

            def wbody(carry):
                c, cum, _ = carry
                v = hist_v[pl.ds(c * LANES, LANES)]
                hist_v[pl.ds(c * LANES, LANES)] = zeros_i
                return c - 1, cum + jnp.sum(v), cum
            c, cum, prev = lax.while_loop(
                wcond, wbody,
                (jnp.int32(nbuckets - 1), jnp.int32(0), jnp.int32(0)))
            d = c + 1

            def zb(b, _):
                hist_v[pl.ds(b * LANES, LANES)] = zeros_i
                return 0
            lax.fori_loop(0, d, zb, 0)
            return d, prev, cum - prev

        def full_round(row_v, shift, pm, pb, krem, masked):
            def hb(o, _):
                for s in range(8):
                    u = _to_ord(row_v[pl.ds((o * 8 + s) * LANES, LANES)])
                    digit = ((u >> shift) & jnp.uint32(0xFF)).astype(jnp.int32)
                    idx = digit * LANES + lane_iota
                    if masked:
                        matc = (u & pm) == pb
                        plsc.addupdate_scatter(hist_v, [idx], ones_i,
                                               mask=matc)
                    else:
                        plsc.addupdate_scatter(hist_v, [idx], ones_i)
                return 0
            lax.fori_loop(0, nchunks // 8, hb, 0)

            d, ca, ceq = find_top(krem, HIST8)
            pb = pb | (d.astype(jnp.uint32) << shift)
            pm = pm | (jnp.uint32(0xFF) << shift)
            krem = krem - ca
            return pm, pb, krem, ceq

        def process_row(r, row_v):
            T = top2_scan(row_v)
            c_t = compact(row_v, T)

            def fast(_):
                return (jnp.uint32(0), jnp.uint32(0), jnp.int32(TOPK), c_t,
                        jnp.int32(8))

            def slow(_):
                pm, pb, krem, ceq = full_round(
                    row_v, jnp.uint32(24), jnp.uint32(0), jnp.uint32(0),
                    jnp.int32(TOPK), masked=False)

                def esc_body(rnd, carry):
                    pm, pb, krem, ceq = carry

                    def run(_):
                        shift = (jnp.uint32(24)
                                 - jnp.uint32(8) * rnd.astype(jnp.uint32))
                        return full_round(row_v, shift, pm, pb, krem,
                                          masked=True)
                    return lax.cond(
                        (TOPK - krem) + ceq > CAP, run,
                        lambda _: (pm, pb, krem, ceq), 0)
                pm, pb, krem, ceq = lax.fori_loop(
                    1, 4, esc_body, (pm, pb, krem, ceq))

                compact(row_v, pb)
                cnt = (TOPK - krem) + ceq
                nrounds = ((jnp.uint32(4) - _popcount_bytes(pm))
                           * jnp.uint32(2)).astype(jnp.int32)
                return pm, pb, krem, cnt, nrounds

            pm, pb, krem, cnt_c, nrounds = lax.cond(
                c_t <= CAP, fast, slow, 0)
            allfull = nrounds == 0
            cchunks = (cnt_c + LANES - 1) // LANES
            cnt_v = jnp.broadcast_to(cnt_c, (LANES,))
            nbits = jnp.uint32(4) * nrounds.astype(jnp.uint32)

            def cr_body(i, carry):
                pm2, pb2, krem2 = carry
                shift = nbits - jnp.uint32(4) * (i.astype(jnp.uint32)
                                                 + jnp.uint32(1))

                def chb(j, _):
                    idx = cand_i[pl.ds(j * LANES, LANES)]
                    valid = (j * LANES + lane_iota) < cnt_v
                    xg = plsc.load_gather(row_v, [idx], mask=valid)
                    u = _to_ord(xg)
                    matc = jnp.logical_and(valid, (u & pm2) == pb2)
                    digit = ((u >> shift) & jnp.uint32(0xF)).astype(jnp.int32)
                    plsc.addupdate_scatter(
                        hist_v, [digit * LANES + lane_iota], ones_i,
                        mask=matc)
                    return 0
                lax.fori_loop(0, cchunks, chb, 0)

                d, excl, _ = find_top(krem2, HIST4)
                pb2 = pb2 | (d.astype(jnp.uint32) << shift)
                pm2 = pm2 | (jnp.uint32(0xF) << shift)
                krem2 = krem2 - excl
                return pm2, pb2, krem2

            _, t, m = lax.fori_loop(0, nrounds, cr_body, (pm, pb, krem))
            t_v = jnp.broadcast_to(t, (LANES,))
            m_v = jnp.broadcast_to(m, (LANES,))

            def emit_scatter(_):
                def sb(i, eqrun):
                    idx = cand_i[pl.ds(i * LANES, LANES)]
                    valid = (i * LANES + lane_iota) < cnt_v
                    xg = plsc.load_gather(row_v, [idx], mask=valid)
                    u = _to_ord(xg)
                    gt = jnp.logical_and(u > t_v, valid)
                    eq = jnp.logical_and(u == t_v, valid)
                    cs = jnp.cumsum(eq.astype(jnp.int32))
                    keep = jnp.logical_or(
                        gt, jnp.logical_and(eq, (cs + eqrun) <= m_v))
                    plsc.store_scatter(out_v, [idx], xg, mask=keep)
                    return eqrun + jnp.max(cs)
                lax.fori_loop(0, cchunks, sb, jnp.int32(0))
                return 0

            def emit_scan(_):
                def ob(i, eqrun):
                    xv = row_v[pl.ds(i * LANES, LANES)]
                    u = _to_ord(xv)
                    gt = u > t_v
                    eq = u == t_v
                    cs = jnp.cumsum(eq.astype(jnp.int32))
                    keep = jnp.logical_or(
                        gt, jnp.logical_and(eq, (cs + eqrun) <= m_v))
                    out_v[pl.ds(i * LANES, LANES)] = jnp.where(
                        keep, xv, zeros_f)
                    return eqrun + jnp.max(cs)
                lax.fori_loop(0, nchunks, ob, jnp.int32(0))
                return 0

            lax.cond(allfull, emit_scan, emit_scatter, 0)
            pltpu.sync_copy(out_v, out_hbm.at[r])

            def restore_scatter(_):
                def rb(i, _):
                    idx = cand_i[pl.ds(i * LANES, LANES)]
                    valid = (i * LANES + lane_iota) < cnt_v
                    plsc.store_scatter(out_v, [idx], zeros_f, mask=valid)
                    return 0
                lax.fori_loop(0, cchunks, rb, 0)
                return 0

            def restore_all(_):
                def zb(i, _):
                    out_v[pl.ds(i * LANES, LANES)] = zeros_f
                    return 0
                lax.fori_loop(0, nchunks, zb, 0)
                return 0

            lax.cond(allfull, restore_all, restore_scatter, 0)

        r0 = wid * rows_per_worker
        cp_a = pltpu.async_copy(x_hbm.at[r0], row_a, sem_a)
        cp_b = pltpu.async_copy(x_hbm.at[r0 + 1], row_b, sem_b)

        def zout(i, _):
            out_v[pl.ds(i * LANES, LANES)] = zeros_f
            return 0
        lax.fori_loop(0, nchunks, zout, 0)

        def zhist(i, _):
            hist_v[pl.ds(i * LANES, LANES)] = zeros_i
            return 0
        lax.fori_loop(0, (LANES * HIST8) // LANES, zhist, 0)

        def zcand(i, _):
            cand_i[pl.ds(i * LANES, LANES)] = zeros_i
            return 0
        lax.fori_loop(0, (CAP + 144) // LANES, zcand, 0)

        cp_a.wait()
        process_row(r0, row_a)
        cp_b.wait()
        process_row(r0 + 1, row_b)

    return topk_kernel


@jax.jit
def kernel(x):
    nrows, ncols = x.shape
    return _make_topk_kernel(nrows, ncols)(x)
